# baseline (speedup 1.0000x reference)
import numpy as np
import jax
import jax.numpy as jnp
from jax import lax
from jax.experimental import pallas as pl
from jax.experimental.pallas import tpu as pltpu

B = 128
V = 100000
VC = 20000
NCHUNK = V // VC
SR = 80
NST = VC // SR
SUB = 8
LANES = 128

TINY = np.float32(np.finfo(np.float32).tiny)
NEG_INF = np.float32(-np.inf)
INT_MAX = np.int32(np.iinfo(np.int32).max)

_K1 = np.int32(42)
_K2 = np.int32(np.uint32(0) ^ np.uint32(42) ^ np.uint32(0x1BD11BDA))


def _rotl(x, r):
    return lax.shift_left(x, np.int32(r)) | lax.shift_right_logical(
        x, np.int32(32 - r))


def _threefry_bits(x1):
    x0 = x1
    x1 = _rotl(x1, 13) ^ x0
    for r in (15, 26, 6):
        x0 = x0 + x1
        x1 = _rotl(x1, r) ^ x0
    x0 = x0 + _K1
    x1 = x1 + np.int32(_K2 + 1)
    for r in (17, 29, 16, 24):
        x0 = x0 + x1
        x1 = _rotl(x1, r) ^ x0
    x0 = x0 + _K2
    x1 = x1 + np.int32(2)
    for r in (13, 15, 26, 6):
        x0 = x0 + x1
        x1 = _rotl(x1, r) ^ x0
    x1 = x1 + np.int32(_K1 + 3)
    for r in (17, 29, 16, 24):
        x0 = x0 + x1
        x1 = _rotl(x1, r) ^ x0
    x0 = x0 + _K1
    x1 = x1 + np.int32(_K2 + 4)
    for r in (13, 15, 26, 6):
        x0 = x0 + x1
        x1 = _rotl(x1, r) ^ x0
    x0 = x0 + _K2
    x1 = x1 + np.int32(5)
    return x0 ^ x1


def _body(lt_ref, out_ref, bv_ref, bi_ref):
    cblk = pl.program_id(0)
    v0 = cblk * VC

    @pl.when(cblk == 0)
    def _init():
        bv_ref[...] = jnp.full((SUB, LANES), NEG_INF, jnp.float32)
        bi_ref[...] = jnp.zeros((SUB, LANES), jnp.int32)

    ctr_base = lax.broadcasted_iota(jnp.int32, (SR, LANES), 1) * np.int32(
        V) + lax.broadcasted_iota(jnp.int32, (SR, LANES), 0)
    sub8 = lax.broadcasted_iota(jnp.int32, (SUB, LANES), 0)

    bv = bv_ref[...]
    bi = bi_ref[...]
    for st in range(NST):
        row = st * SR
        bits = _threefry_bits(ctr_base + (v0 + np.int32(row + 42)))
        fb = lax.shift_right_logical(bits, np.int32(9)) | np.int32(0x3F800000)
        f = lax.bitcast_convert_type(fb, jnp.float32) - np.float32(1.0)
        e = -jnp.log(f)
        vals = lt_ref[row:row + SR, :] - jnp.log(e)
        for k in range(SR // SUB):
            slab = vals[k * SUB:(k + 1) * SUB, :]
            sidx = sub8 + (v0 + np.int32(row + k * SUB))
            better = slab > bv
            bi = jnp.where(better, sidx, bi)
            bv = jnp.where(better, slab, bv)
    bv_ref[...] = bv
    bi_ref[...] = bi

    @pl.when(cblk == NCHUNK - 1)
    def _finalize():
        m = jnp.max(bv_ref[...], axis=0, keepdims=True)
        cand = jnp.where(bv_ref[...] == m, bi_ref[...], INT_MAX)
        out_ref[...] = jnp.min(cand, axis=0, keepdims=True)


@jax.jit
def kernel(logits):
    lt = logits.T
    out = pl.pallas_call(
        _body,
        grid=(NCHUNK,),
        in_specs=[pl.BlockSpec((VC, LANES), lambda c: (c, 0))],
        out_specs=pl.BlockSpec((1, LANES), lambda c: (0, 0)),
        out_shape=jax.ShapeDtypeStruct((1, LANES), jnp.int32),
        scratch_shapes=[
            pltpu.VMEM((SUB, LANES), jnp.float32),
            pltpu.VMEM((SUB, LANES), jnp.int32),
        ],
        compiler_params=pltpu.CompilerParams(
            dimension_semantics=("arbitrary",)),
    )(lt)
    return out.reshape(B).astype(jnp.int64)


# speedup vs baseline: 1.3279x; 1.3279x over previous
import numpy as np
import jax
import jax.numpy as jnp
from jax import lax
from jax.experimental import pallas as pl
from jax.experimental.pallas import tpu as pltpu

B = 128
V = 100000
VC = 10000
NCHUNK = V // VC
SR = 80
NST = VC // SR
SUB = 8
LANES = 128

TINY = np.float32(np.finfo(np.float32).tiny)
NEG_INF = np.float32(-np.inf)
INT_MAX = np.int32(np.iinfo(np.int32).max)

_K1 = np.int32(42)
_K2 = np.int32(np.uint32(0) ^ np.uint32(42) ^ np.uint32(0x1BD11BDA))


def _rotl(x, r):
    return lax.shift_left(x, np.int32(r)) | lax.shift_right_logical(
        x, np.int32(32 - r))


def _threefry_bits(x1):
    x0 = x1
    x1 = _rotl(x1, 13) ^ x0
    for r in (15, 26, 6):
        x0 = x0 + x1
        x1 = _rotl(x1, r) ^ x0
    x0 = x0 + _K1
    x1 = x1 + np.int32(_K2 + 1)
    for r in (17, 29, 16, 24):
        x0 = x0 + x1
        x1 = _rotl(x1, r) ^ x0
    x0 = x0 + _K2
    x1 = x1 + np.int32(2)
    for r in (13, 15, 26, 6):
        x0 = x0 + x1
        x1 = _rotl(x1, r) ^ x0
    x1 = x1 + np.int32(_K1 + 3)
    for r in (17, 29, 16, 24):
        x0 = x0 + x1
        x1 = _rotl(x1, r) ^ x0
    x0 = x0 + _K1
    x1 = x1 + np.int32(_K2 + 4)
    for r in (13, 15, 26, 6):
        x0 = x0 + x1
        x1 = _rotl(x1, r) ^ x0
    x0 = x0 + _K2
    x1 = x1 + np.int32(5)
    return x0 ^ x1


def _body(lt_ref, out_ref, bv_ref, bi_ref):
    cblk = pl.program_id(0)
    v0 = cblk * VC

    @pl.when(cblk == 0)
    def _init():
        bv_ref[...] = jnp.full((SUB, LANES), NEG_INF, jnp.float32)
        bi_ref[...] = jnp.zeros((SUB, LANES), jnp.int32)

    ctr_base = lax.broadcasted_iota(jnp.int32, (SR, LANES), 1) * np.int32(
        V) + lax.broadcasted_iota(jnp.int32, (SR, LANES), 0)
    sub8 = lax.broadcasted_iota(jnp.int32, (SUB, LANES), 0)

    bv = bv_ref[...]
    bi = bi_ref[...]
    for st in range(NST):
        row = st * SR
        bits = _threefry_bits(ctr_base + (v0 + np.int32(row + 42)))
        fb = lax.shift_right_logical(bits, np.int32(9)) | np.int32(0x3F800000)
        f = lax.bitcast_convert_type(fb, jnp.float32) - np.float32(1.0)
        e = -jnp.log(f)
        vals = lt_ref[row:row + SR, :] - jnp.log(e)
        for k in range(SR // SUB):
            slab = vals[k * SUB:(k + 1) * SUB, :]
            sidx = sub8 + (v0 + np.int32(row + k * SUB))
            better = slab > bv
            bi = jnp.where(better, sidx, bi)
            bv = jnp.where(better, slab, bv)
    bv_ref[...] = bv
    bi_ref[...] = bi

    @pl.when(cblk == NCHUNK - 1)
    def _finalize():
        m = jnp.max(bv_ref[...], axis=0, keepdims=True)
        cand = jnp.where(bv_ref[...] == m, bi_ref[...], INT_MAX)
        out_ref[...] = jnp.min(cand, axis=0, keepdims=True)


@jax.jit
def kernel(logits):
    lt = logits.T
    out = pl.pallas_call(
        _body,
        grid=(NCHUNK,),
        in_specs=[pl.BlockSpec((VC, LANES), lambda c: (c, 0))],
        out_specs=pl.BlockSpec((1, LANES), lambda c: (0, 0)),
        out_shape=jax.ShapeDtypeStruct((1, LANES), jnp.int32),
        scratch_shapes=[
            pltpu.VMEM((SUB, LANES), jnp.float32),
            pltpu.VMEM((SUB, LANES), jnp.int32),
        ],
        compiler_params=pltpu.CompilerParams(
            dimension_semantics=("arbitrary",)),
    )(lt)
    return out.reshape(B).astype(jnp.int64)


# speedup vs baseline: 1.3396x; 1.0088x over previous
import numpy as np
import jax
import jax.numpy as jnp
from jax import lax
from jax.experimental import pallas as pl
from jax.experimental.pallas import tpu as pltpu

B = 128
V = 100000
VC = 10000
NCHUNK = V // VC
SR = 80
NST = VC // SR
SUB = 8
LANES = 128

TINY = np.float32(np.finfo(np.float32).tiny)
NEG_INF = np.float32(-np.inf)
INT_MAX = np.int32(np.iinfo(np.int32).max)

_K1 = np.int32(42)
_K2 = np.int32(np.uint32(0) ^ np.uint32(42) ^ np.uint32(0x1BD11BDA))


def _rotl(x, r):
    return lax.shift_left(x, np.int32(r)) | lax.shift_right_logical(
        x, np.int32(32 - r))


def _threefry_bits(x1):
    x0 = x1
    x1 = _rotl(x1, 13) ^ x0
    for r in (15, 26, 6):
        x0 = x0 + x1
        x1 = _rotl(x1, r) ^ x0
    x0 = x0 + _K1
    x1 = x1 + np.int32(_K2 + 1)
    for r in (17, 29, 16, 24):
        x0 = x0 + x1
        x1 = _rotl(x1, r) ^ x0
    x0 = x0 + _K2
    x1 = x1 + np.int32(2)
    for r in (13, 15, 26, 6):
        x0 = x0 + x1
        x1 = _rotl(x1, r) ^ x0
    x1 = x1 + np.int32(_K1 + 3)
    for r in (17, 29, 16, 24):
        x0 = x0 + x1
        x1 = _rotl(x1, r) ^ x0
    x0 = x0 + _K1
    x1 = x1 + np.int32(_K2 + 4)
    for r in (13, 15, 26, 6):
        x0 = x0 + x1
        x1 = _rotl(x1, r) ^ x0
    x0 = x0 + _K2
    x1 = x1 + np.int32(5)
    return x0 ^ x1


def _body(lt_ref, out_ref, bv_ref, bi_ref):
    cblk = pl.program_id(0)
    v0 = cblk * VC

    @pl.when(cblk == 0)
    def _init():
        bv_ref[...] = jnp.full((SUB, LANES), NEG_INF, jnp.float32)
        bi_ref[...] = jnp.zeros((SUB, LANES), jnp.int32)

    ctr_base = lax.broadcasted_iota(jnp.int32, (SR, LANES), 1) * np.int32(
        V) + lax.broadcasted_iota(jnp.int32, (SR, LANES), 0)
    sub8 = lax.broadcasted_iota(jnp.int32, (SUB, LANES), 0)

    bv = bv_ref[...]
    bi = bi_ref[...]
    for st in range(NST):
        row = st * SR
        bits = _threefry_bits(ctr_base + (v0 + np.int32(row + 42)))
        fb = lax.shift_right_logical(bits, np.int32(9)) | np.int32(0x3F800000)
        f = lax.bitcast_convert_type(fb, jnp.float32) - np.float32(1.0)
        e = -jnp.log(f)
        vals = lt_ref[row:row + SR, :] - jnp.log(e)
        for k in range(SR // SUB):
            slab = vals[k * SUB:(k + 1) * SUB, :]
            sidx = sub8 + (v0 + np.int32(row + k * SUB))
            better = slab > bv
            bi = jnp.where(better, sidx, bi)
            bv = jnp.maximum(bv, slab)
    bv_ref[...] = bv
    bi_ref[...] = bi

    @pl.when(cblk == NCHUNK - 1)
    def _finalize():
        m = jnp.max(bv_ref[...], axis=0, keepdims=True)
        cand = jnp.where(bv_ref[...] == m, bi_ref[...], INT_MAX)
        out_ref[...] = jnp.min(cand, axis=0, keepdims=True)


@jax.jit
def kernel(logits):
    lt = logits.T
    out = pl.pallas_call(
        _body,
        grid=(NCHUNK,),
        in_specs=[pl.BlockSpec((VC, LANES), lambda c: (c, 0))],
        out_specs=pl.BlockSpec((1, LANES), lambda c: (0, 0)),
        out_shape=jax.ShapeDtypeStruct((1, LANES), jnp.int32),
        scratch_shapes=[
            pltpu.VMEM((SUB, LANES), jnp.float32),
            pltpu.VMEM((SUB, LANES), jnp.int32),
        ],
        compiler_params=pltpu.CompilerParams(
            dimension_semantics=("arbitrary",)),
    )(lt)
    return out.reshape(B).astype(jnp.int64)
